# baseline (speedup 1.0000x reference)
import functools

import jax
import jax.numpy as jnp
from jax import lax
from jax.experimental import pallas as pl
from jax.experimental.pallas import tpu as pltpu
from jax.experimental.pallas import tpu_sc as plsc

BATCH = 16384
EMB_DIM = 64
NC = 2
NS = 16
NW = NC * NS
B_PER_W = BATCH // NW
CH = 128
NCH = B_PER_W // CH
ID_ROWS = BATCH // CH

_sc_mesh = plsc.VectorSubcoreMesh(core_axis_name="c", subcore_axis_name="s")


@functools.partial(
    pl.kernel,
    mesh=_sc_mesh,
    out_type=[
        jax.ShapeDtypeStruct((BATCH, EMB_DIM), jnp.float32),
        jax.ShapeDtypeStruct((BATCH, EMB_DIM), jnp.float32),
    ],
    scratch_types=[
        pltpu.VMEM((NCH, CH), jnp.int32),
        pltpu.VMEM((NCH, CH), jnp.int32),
        pltpu.VMEM((B_PER_W, EMB_DIM), jnp.float32),
        pltpu.VMEM((B_PER_W, EMB_DIM), jnp.float32),
        pltpu.SemaphoreType.DMA,
    ],
    compiler_params=pltpu.CompilerParams(use_tc_tiling_on_sc=False),
)
def _sc_gather(uid_hbm, iid_hbm, ut_hbm, it_hbm, u_out, i_out,
               uidx_v, iidx_v, urows_v, irows_v, sem):
    wid = lax.axis_index("s") * NC + lax.axis_index("c")
    rbase = wid * NCH
    base = wid * B_PER_W
    pltpu.sync_copy(uid_hbm.at[pl.ds(rbase, NCH)], uidx_v)
    pltpu.sync_copy(iid_hbm.at[pl.ds(rbase, NCH)], iidx_v)
    copies = []
    for c in range(NCH):
        copies.append(
            pltpu.async_copy(ut_hbm.at[uidx_v.at[c]],
                             urows_v.at[pl.ds(c * CH, CH)], sem))
        copies.append(
            pltpu.async_copy(it_hbm.at[iidx_v.at[c]],
                             irows_v.at[pl.ds(c * CH, CH)], sem))
    for cp in copies:
        cp.wait()
    pltpu.sync_copy(urows_v, u_out.at[pl.ds(base, B_PER_W)])
    pltpu.sync_copy(irows_v, i_out.at[pl.ds(base, B_PER_W)])


MLP_BLK = 2048


def _mlp_body(u_ref, i_ref, w1u_ref, w1i_ref, b1_ref, w2t_ref, b2_ref,
              w3_ref, b3_ref, o_ref):
    h = jnp.dot(u_ref[...], w1u_ref[...], preferred_element_type=jnp.float32)
    h = h + jnp.dot(i_ref[...], w1i_ref[...], preferred_element_type=jnp.float32)
    h = jnp.maximum(h + b1_ref[...], 0.0)
    h2 = jnp.dot(h, w2t_ref[...], preferred_element_type=jnp.float32)
    h2 = jnp.maximum(h2 + b2_ref[...], 0.0)
    o_ref[...] = jnp.sum(h2 * w3_ref[...], axis=1) + b3_ref[0, 0]


def _mlp(u_emb, i_emb, w1u, w1i, b1, w2t, b2, w3, b3):
    grid = (BATCH // MLP_BLK,)
    full = lambda shape: pl.BlockSpec(shape, lambda i: (0, 0))
    return pl.pallas_call(
        _mlp_body,
        grid=grid,
        in_specs=[
            pl.BlockSpec((MLP_BLK, EMB_DIM), lambda i: (i, 0)),
            pl.BlockSpec((MLP_BLK, EMB_DIM), lambda i: (i, 0)),
            full((EMB_DIM, 128)),
            full((EMB_DIM, 128)),
            full((1, 128)),
            full((128, 64)),
            full((1, 64)),
            full((1, 64)),
            full((1, 1)),
        ],
        out_specs=pl.BlockSpec((MLP_BLK,), lambda i: (i,)),
        out_shape=jax.ShapeDtypeStruct((BATCH,), jnp.float32),
    )(u_emb, i_emb, w1u, w1i, b1, w2t, b2, w3, b3)


def kernel(user_ids, item_ids, user_table, item_table, W1, b1, W2, b2, W3, b3):
    uid = user_ids.astype(jnp.int32).reshape(ID_ROWS, CH)
    iid = item_ids.astype(jnp.int32).reshape(ID_ROWS, CH)
    u_emb, i_emb = _sc_gather(uid, iid, user_table, item_table)
    w1u = W1[:, :EMB_DIM].T
    w1i = W1[:, EMB_DIM:].T
    return _mlp(u_emb, i_emb, w1u, w1i, b1.reshape(1, 128), W2.T,
                b2.reshape(1, 64), W3, b3.reshape(1, 1))


# speedup vs baseline: 1.0037x; 1.0037x over previous
import functools

import jax
import jax.numpy as jnp
from jax import lax
from jax.experimental import pallas as pl
from jax.experimental.pallas import tpu as pltpu
from jax.experimental.pallas import tpu_sc as plsc

BATCH = 16384
EMB_DIM = 64
NC = 2
NS = 16
NW = NC * NS
B_PER_W = BATCH // NW
CH = 128
NCH = B_PER_W // CH
HALF = NCH // 2
HC = HALF * CH
ID_ROWS = BATCH // CH

_sc_mesh = plsc.VectorSubcoreMesh(core_axis_name="c", subcore_axis_name="s")


@functools.partial(
    pl.kernel,
    mesh=_sc_mesh,
    out_type=[
        jax.ShapeDtypeStruct((BATCH, 128), jnp.float32),
        jax.ShapeDtypeStruct((BATCH, 128), jnp.float32),
    ],
    scratch_types=[
        pltpu.VMEM((2 * NCH, CH), jnp.int32),
        pltpu.VMEM((2 * NCH, CH), jnp.int32),
        pltpu.VMEM((HC, 128), jnp.float32),
        pltpu.VMEM((HC, 128), jnp.float32),
        pltpu.SemaphoreType.DMA,
    ],
)
def _sc_gather(uid_hbm, iid_hbm, ut_hbm, it_hbm, u_out, i_out,
               uidx_v, iidx_v, ubuf_v, ibuf_v, sem):
    wid = lax.axis_index("s") * NC + lax.axis_index("c")
    base = wid * B_PER_W
    pltpu.sync_copy(uid_hbm.at[pl.ds((wid // 2) * 2 * NCH, 2 * NCH)], uidx_v)
    pltpu.sync_copy(iid_hbm.at[pl.ds((wid // 2) * 2 * NCH, 2 * NCH)], iidx_v)
    for h in range(NCH // HALF):
        copies = []
        for c in range(HALF):
            row = (wid % 2) * NCH + h * HALF + c
            copies.append(
                pltpu.async_copy(ut_hbm.at[uidx_v.at[row]],
                                 ubuf_v.at[pl.ds(c * CH, CH)], sem))
            copies.append(
                pltpu.async_copy(it_hbm.at[iidx_v.at[row]],
                                 ibuf_v.at[pl.ds(c * CH, CH)], sem))
        for cp in copies:
            cp.wait()
        pltpu.sync_copy(ubuf_v, u_out.at[pl.ds(base + h * HC, HC)])
        pltpu.sync_copy(ibuf_v, i_out.at[pl.ds(base + h * HC, HC)])


MLP_BLK = 2048


def _mlp_body(u_ref, i_ref, pu_ref, pi_ref, w1u_ref, w1i_ref, b1_ref,
              w2t_ref, b2_ref, w3_ref, b3_ref, o_ref):
    xu = u_ref[...]
    xi = i_ref[...]
    pu = pu_ref[...]
    pi = pi_ref[...]
    u = xu[:, :EMB_DIM] + pu * (xu[:, EMB_DIM:] - xu[:, :EMB_DIM])
    it = xi[:, :EMB_DIM] + pi * (xi[:, EMB_DIM:] - xi[:, :EMB_DIM])
    h = jnp.dot(u, w1u_ref[...], preferred_element_type=jnp.float32)
    h = h + jnp.dot(it, w1i_ref[...], preferred_element_type=jnp.float32)
    h = jnp.maximum(h + b1_ref[...], 0.0)
    h2 = jnp.dot(h, w2t_ref[...], preferred_element_type=jnp.float32)
    h2 = jnp.maximum(h2 + b2_ref[...], 0.0)
    o_ref[...] = jnp.sum(h2 * w3_ref[...], axis=1) + b3_ref[0, 0]


def _mlp(u_raw, i_raw, pu, pi, w1u, w1i, b1, w2t, b2, w3, b3):
    grid = (BATCH // MLP_BLK,)
    full = lambda shape: pl.BlockSpec(shape, lambda i: (0, 0))
    return pl.pallas_call(
        _mlp_body,
        grid=grid,
        in_specs=[
            pl.BlockSpec((MLP_BLK, 128), lambda i: (i, 0)),
            pl.BlockSpec((MLP_BLK, 128), lambda i: (i, 0)),
            pl.BlockSpec((MLP_BLK, 1), lambda i: (i, 0)),
            pl.BlockSpec((MLP_BLK, 1), lambda i: (i, 0)),
            full((EMB_DIM, 128)),
            full((EMB_DIM, 128)),
            full((1, 128)),
            full((128, 64)),
            full((1, 64)),
            full((1, 64)),
            full((1, 1)),
        ],
        out_specs=pl.BlockSpec((MLP_BLK,), lambda i: (i,)),
        out_shape=jax.ShapeDtypeStruct((BATCH,), jnp.float32),
    )(u_raw, i_raw, pu, pi, w1u, w1i, b1, w2t, b2, w3, b3)


def kernel(user_ids, item_ids, user_table, item_table, W1, b1, W2, b2, W3, b3):
    uid = user_ids.astype(jnp.int32)
    iid = item_ids.astype(jnp.int32)
    uid_pair = (uid >> 1).reshape(ID_ROWS, CH)
    iid_pair = (iid >> 1).reshape(ID_ROWS, CH)
    pu = (uid & 1).astype(jnp.float32).reshape(BATCH, 1)
    pi = (iid & 1).astype(jnp.float32).reshape(BATCH, 1)
    ut2 = user_table.reshape(user_table.shape[0] // 2, 128)
    it2 = item_table.reshape(item_table.shape[0] // 2, 128)
    u_raw, i_raw = _sc_gather(uid_pair, iid_pair, ut2, it2)
    w1u = W1[:, :EMB_DIM].T
    w1i = W1[:, EMB_DIM:].T
    return _mlp(u_raw, i_raw, pu, pi, w1u, w1i, b1.reshape(1, 128), W2.T,
                b2.reshape(1, 64), W3, b3.reshape(1, 1))
